# baseline (speedup 1.0000x reference)
import functools

import jax
import jax.numpy as jnp
from jax import lax
from jax.experimental import pallas as pl
from jax.experimental.pallas import tpu as pltpu
from jax.experimental.pallas import tpu_sc as plsc

N = 10000
E = 320000
D = 128
DE = 16

NC = 2
NS = 16
NW = NC * NS
EPW = E // NW
CB = 80
NCH = EPW // CB
NP = 10240
RPT = NP // NS
ZR = 128


def _proj_body(h_ref, ws_ref, wt_ref, os_ref, ot_ref):
    x = h_ref[...]
    os_ref[...] = jnp.dot(x, ws_ref[...], preferred_element_type=jnp.float32)
    ot_ref[...] = jnp.dot(x, wt_ref[...], preferred_element_type=jnp.float32)


def _project(h, w1s, w1t):
    bn = 1000
    return pl.pallas_call(
        _proj_body,
        grid=(N // bn,),
        in_specs=[
            pl.BlockSpec((bn, D), lambda i: (i, 0)),
            pl.BlockSpec((D, D), lambda i: (0, 0)),
            pl.BlockSpec((D, D), lambda i: (0, 0)),
        ],
        out_specs=[
            pl.BlockSpec((bn, D), lambda i: (i, 0)),
            pl.BlockSpec((bn, D), lambda i: (i, 0)),
        ],
        out_shape=[
            jax.ShapeDtypeStruct((N, D), jnp.float32),
            jax.ShapeDtypeStruct((N, D), jnp.float32),
        ],
    )(h, w1s, w1t)


DA = D + 16


def _sc_gather_body(hs_hbm, ht_hbm, row_hbm, col_hbm, w1r_hbm,
                    pre_hbm,
                    rowv, colv, w1rv, bufs0, buft0, bufs1, buft1, ob0, ob1,
                    ss0, st0, ss1, st1, sw0, sw1):
    c = lax.axis_index("c")
    s = lax.axis_index("s")
    wid = s * NC + c
    base = wid * EPW

    pltpu.sync_copy(row_hbm.at[pl.ds(base, EPW)], rowv)
    pltpu.sync_copy(col_hbm.at[pl.ds(base, EPW)], colv)
    pltpu.sync_copy(w1r_hbm, w1rv)
    wv = [w1rv[pl.ds(k * 16, 16)] for k in range(D // 16)]
    OB = (ob0, ob1)

    BS = (bufs0, bufs1)
    BT = (buft0, buft1)
    SS = (ss0, ss1)
    ST = (st0, st1)
    SW = (sw0, sw1)

    def issue_gathers(j, sel):
        off = j * CB
        pltpu.async_copy(hs_hbm.at[rowv.at[pl.ds(off, CB)]], BS[sel], SS[sel])
        pltpu.async_copy(ht_hbm.at[colv.at[pl.ds(off, CB)]], BT[sel], ST[sel])

    issue_gathers(0, 0)

    def chunk(j, carry):
        for sel in (0, 1):
            nsel = 1 - sel

            @pl.when(j % 2 == sel)
            def _():
                @pl.when(j >= 1)
                def _():
                    pltpu.make_async_copy(
                        OB[nsel], pre_hbm.at[pl.ds(0, CB)], SW[nsel]).wait()

                @pl.when(j + 1 < NCH)
                def _():
                    issue_gathers(j + 1, nsel)

                pltpu.make_async_copy(
                    hs_hbm.at[rowv.at[pl.ds(0, CB)]], BS[sel], SS[sel]).wait()
                pltpu.make_async_copy(
                    ht_hbm.at[colv.at[pl.ds(0, CB)]], BT[sel], ST[sel]).wait()

                def add_body(r, carry2):
                    p = (BS[sel][r, pl.ds(D, 16)] * BT[sel][r, pl.ds(D, 16)])
                    rad = p[0] + p[1] + p[2] + p[3] + p[4]
                    for cc in range(D // 16):
                        sl = pl.ds(cc * 16, 16)
                        OB[sel][r, sl] = (BS[sel][r, sl] + BT[sel][r, sl]
                                          + rad * wv[cc])
                    return carry2

                lax.fori_loop(0, CB, add_body, 0)
                pltpu.async_copy(
                    OB[sel], pre_hbm.at[pl.ds(base + j * CB, CB)], SW[sel])
        return carry

    lax.fori_loop(0, NCH, chunk, 0)
    last = (NCH - 1) % 2
    pltpu.make_async_copy(OB[last], pre_hbm.at[pl.ds(0, CB)], SW[last]).wait()


def _sc_gather(hsa, hta, row, col, w1r_flat):
    kern = functools.partial(
        pl.kernel,
        out_type=jax.ShapeDtypeStruct((E, D), jnp.float32),
        mesh=plsc.VectorSubcoreMesh(core_axis_name="c", subcore_axis_name="s"),
        compiler_params=pltpu.CompilerParams(use_tc_tiling_on_sc=False),
        scratch_types=[
            pltpu.VMEM((EPW,), jnp.int32),
            pltpu.VMEM((EPW,), jnp.int32),
            pltpu.VMEM((D,), jnp.float32),
            pltpu.VMEM((CB, DA), jnp.float32),
            pltpu.VMEM((CB, DA), jnp.float32),
            pltpu.VMEM((CB, DA), jnp.float32),
            pltpu.VMEM((CB, DA), jnp.float32),
            pltpu.VMEM((CB, D), jnp.float32),
            pltpu.VMEM((CB, D), jnp.float32),
            pltpu.SemaphoreType.DMA,
            pltpu.SemaphoreType.DMA,
            pltpu.SemaphoreType.DMA,
            pltpu.SemaphoreType.DMA,
            pltpu.SemaphoreType.DMA,
            pltpu.SemaphoreType.DMA,
        ],
    )(_sc_gather_body)
    return kern(hsa, hta, row, col, w1r_flat)


def _edge_body(pre_ref, ea_ref, w1e_ref, b1_ref,
               w2_ref, b2_ref, out_ref):
    x = (pre_ref[...]
         + jnp.dot(ea_ref[...], w1e_ref[...], preferred_element_type=jnp.float32)
         + b1_ref[...])
    m = x * jax.nn.sigmoid(x)
    y = jnp.dot(m, w2_ref[...], preferred_element_type=jnp.float32) + b2_ref[...]
    out_ref[...] = y * jax.nn.sigmoid(y)


def _edge_mlp(pre, ea, w1e, b1, w2, b2):
    be = 2000
    return pl.pallas_call(
        _edge_body,
        grid=(E // be,),
        in_specs=[
            pl.BlockSpec((be, D), lambda i: (i, 0)),
            pl.BlockSpec((be, DE), lambda i: (i, 0)),
            pl.BlockSpec((DE, D), lambda i: (0, 0)),
            pl.BlockSpec((1, D), lambda i: (0, 0)),
            pl.BlockSpec((D, D), lambda i: (0, 0)),
            pl.BlockSpec((1, D), lambda i: (0, 0)),
        ],
        out_specs=pl.BlockSpec((be, D), lambda i: (i, 0)),
        out_shape=jax.ShapeDtypeStruct((E, D), jnp.float32),
    )(pre, ea, w1e, b1, w2, b2)


def _sc_scatter_body(mij_hbm, row_hbm, out_hbm,
                     idx0, idx1, mb0, mb1, zbuf, aggsp,
                     si0, si1, sm0, sm1):
    c = lax.axis_index("c")
    s = lax.axis_index("s")
    wid = s * NC + c
    base = wid * EPW

    IDX = (idx0, idx1)
    MB = (mb0, mb1)
    SI = (si0, si1)
    SM = (sm0, sm1)

    def issue_loads(j, sel):
        ebase = base + j * CB
        pltpu.async_copy(row_hbm.at[pl.ds(ebase, CB)], IDX[sel], SI[sel])
        pltpu.async_copy(mij_hbm.at[pl.ds(ebase, CB)], MB[sel], SM[sel])

    def zb(r, carry):
        for cc in range(D // 16):
            zbuf[r, pl.ds(cc * 16, 16)] = jnp.zeros((16,), jnp.float32)
        return carry

    lax.fori_loop(0, ZR, zb, 0)
    for k in range(RPT // ZR):
        pltpu.sync_copy(zbuf, aggsp.at[pl.ds(s * RPT + k * ZR, ZR)])
    plsc.subcore_barrier()

    issue_loads(0, 0)

    def chunk(j, carry):
        for sel in (0, 1):
            nsel = 1 - sel

            @pl.when(j % 2 == sel)
            def _():
                @pl.when(j + 1 < NCH)
                def _():
                    issue_loads(j + 1, nsel)

                pltpu.make_async_copy(
                    row_hbm.at[pl.ds(0, CB)], IDX[sel], SI[sel]).wait()
                pltpu.make_async_copy(
                    mij_hbm.at[pl.ds(0, CB)], MB[sel], SM[sel]).wait()
                pltpu.sync_copy(MB[sel], aggsp.at[IDX[sel]], add=True)
        return carry

    lax.fori_loop(0, NCH, chunk, 0)
    plsc.subcore_barrier()

    for k in range(RPT // ZR):
        r0 = s * RPT + k * ZR
        pltpu.sync_copy(aggsp.at[pl.ds(r0, ZR)], zbuf)
        pltpu.sync_copy(zbuf, out_hbm.at[c, pl.ds(r0, ZR)])


def _sc_scatter(mij, row):
    kern = functools.partial(
        pl.kernel,
        out_type=jax.ShapeDtypeStruct((NC, NP, D), jnp.float32),
        mesh=plsc.VectorSubcoreMesh(core_axis_name="c", subcore_axis_name="s"),
        scratch_types=[
            pltpu.VMEM((CB,), jnp.int32),
            pltpu.VMEM((CB,), jnp.int32),
            pltpu.VMEM((CB, D), jnp.float32),
            pltpu.VMEM((CB, D), jnp.float32),
            pltpu.VMEM((ZR, D), jnp.float32),
            pltpu.VMEM_SHARED((NP, D), jnp.float32),
            pltpu.SemaphoreType.DMA,
            pltpu.SemaphoreType.DMA,
            pltpu.SemaphoreType.DMA,
            pltpu.SemaphoreType.DMA,
        ],
    )(_sc_scatter_body)
    return kern(mij, row)


def _node_body(h_ref, agg_ref, w3h_ref, w3a_ref, b3_ref, w4_ref, b4_ref,
               out_ref):
    hb = h_ref[...]
    agg = agg_ref[0] + agg_ref[1]
    t = (jnp.dot(hb, w3h_ref[...], preferred_element_type=jnp.float32)
         + jnp.dot(agg, w3a_ref[...], preferred_element_type=jnp.float32)
         + b3_ref[...])
    t = t * jax.nn.sigmoid(t)
    out_ref[...] = (hb
                    + jnp.dot(t, w4_ref[...], preferred_element_type=jnp.float32)
                    + b4_ref[...])


def _node_mlp(h, agg2, w3h, w3a, b3, w4, b4):
    bn = 1000
    return pl.pallas_call(
        _node_body,
        grid=(N // bn,),
        in_specs=[
            pl.BlockSpec((bn, D), lambda i: (i, 0)),
            pl.BlockSpec((NC, bn, D), lambda i: (0, i, 0)),
            pl.BlockSpec((D, D), lambda i: (0, 0)),
            pl.BlockSpec((D, D), lambda i: (0, 0)),
            pl.BlockSpec((1, D), lambda i: (0, 0)),
            pl.BlockSpec((D, D), lambda i: (0, 0)),
            pl.BlockSpec((1, D), lambda i: (0, 0)),
        ],
        out_specs=pl.BlockSpec((bn, D), lambda i: (i, 0)),
        out_shape=jax.ShapeDtypeStruct((N, D), jnp.float32),
    )(h, agg2, w3h, w3a, b3, w4, b4)


def kernel(h, edge_index, coord, edge_attr, W1, b1, W2, b2, W3, b3, W4, b4):
    row = edge_index[0].astype(jnp.int32)
    col = edge_index[1].astype(jnp.int32)
    w1s = W1[:D]
    w1t = W1[D:2 * D]
    w1e = W1[2 * D + 1:]

    hs, ht = _project(h, w1s, w1t)
    n2 = jnp.sum(coord * coord, axis=1, keepdims=True)
    ones = jnp.ones((N, 1), jnp.float32)
    zpad = jnp.zeros((N, 11), jnp.float32)
    hsa = jnp.concatenate([hs, coord, ones, n2, zpad], axis=1)
    hta = jnp.concatenate([ht, -2.0 * coord, n2, ones, zpad], axis=1)
    pre = _sc_gather(hsa, hta, row, col, W1[2 * D])
    mij = _edge_mlp(pre, edge_attr, w1e,
                    b1.reshape(1, D), W2, b2.reshape(1, D))
    agg2 = _sc_scatter(mij, row)
    return _node_mlp(h, agg2, W3[:D], W3[D:], b3.reshape(1, D), W4,
                     b4.reshape(1, D))


# speedup vs baseline: 1.1265x; 1.1265x over previous
import functools

import jax
import jax.numpy as jnp
from jax import lax
from jax.experimental import pallas as pl
from jax.experimental.pallas import tpu as pltpu
from jax.experimental.pallas import tpu_sc as plsc

N = 10000
E = 320000
D = 128
DE = 16

NC = 2
NS = 16
NW = NC * NS
EPW = E // NW
CB = 80
NCH = EPW // CB
NP = 10240
RPT = NP // NS
ZR = 128


def _proj_body(h_ref, ws_ref, wt_ref, os_ref, ot_ref):
    x = h_ref[...]
    os_ref[...] = jnp.dot(x, ws_ref[...], preferred_element_type=jnp.float32)
    ot_ref[...] = jnp.dot(x, wt_ref[...], preferred_element_type=jnp.float32)


def _project(h, w1s, w1t):
    bn = 1000
    return pl.pallas_call(
        _proj_body,
        grid=(N // bn,),
        in_specs=[
            pl.BlockSpec((bn, D), lambda i: (i, 0)),
            pl.BlockSpec((D, D), lambda i: (0, 0)),
            pl.BlockSpec((D, D), lambda i: (0, 0)),
        ],
        out_specs=[
            pl.BlockSpec((bn, D), lambda i: (i, 0)),
            pl.BlockSpec((bn, D), lambda i: (i, 0)),
        ],
        out_shape=[
            jax.ShapeDtypeStruct((N, D), jnp.float32),
            jax.ShapeDtypeStruct((N, D), jnp.float32),
        ],
    )(h, w1s, w1t)


DA = D + 16


def _sc_gather_body(hs_hbm, ht_hbm, row_hbm, col_hbm,
                    pre_hbm,
                    rowv, colv, bufs0, buft0, bufs1, buft1,
                    ss0, st0, ss1, st1, sw0, sw1):
    c = lax.axis_index("c")
    s = lax.axis_index("s")
    wid = s * NC + c
    base = wid * EPW

    pltpu.sync_copy(row_hbm.at[pl.ds(base, EPW)], rowv)
    pltpu.sync_copy(col_hbm.at[pl.ds(base, EPW)], colv)

    BS = (bufs0, bufs1)
    BT = (buft0, buft1)
    SS = (ss0, ss1)
    ST = (st0, st1)
    SW = (sw0, sw1)

    def issue_gathers(j, sel):
        off = j * CB
        pltpu.async_copy(hs_hbm.at[rowv.at[pl.ds(off, CB)]], BS[sel], SS[sel])
        pltpu.async_copy(ht_hbm.at[colv.at[pl.ds(off, CB)]], BT[sel], ST[sel])

    issue_gathers(0, 0)

    def chunk(j, carry):
        for sel in (0, 1):
            nsel = 1 - sel

            @pl.when(j % 2 == sel)
            def _():
                @pl.when(j >= 1)
                def _():
                    pltpu.make_async_copy(
                        BS[nsel], pre_hbm.at[pl.ds(0, CB)], SW[nsel]).wait()

                @pl.when(j + 1 < NCH)
                def _():
                    issue_gathers(j + 1, nsel)

                pltpu.make_async_copy(
                    hs_hbm.at[rowv.at[pl.ds(0, CB)]], BS[sel], SS[sel]).wait()
                pltpu.make_async_copy(
                    ht_hbm.at[colv.at[pl.ds(0, CB)]], BT[sel], ST[sel]).wait()

                def add_body(r, carry2):
                    for cc in range(D // 16):
                        sl = pl.ds(cc * 16, 16)
                        BS[sel][r, sl] = BS[sel][r, sl] + BT[sel][r, sl]
                    sl = pl.ds(D, 16)
                    BS[sel][r, sl] = BS[sel][r, sl] * BT[sel][r, sl]
                    return carry2

                lax.fori_loop(0, CB, add_body, 0)
                pltpu.async_copy(
                    BS[sel], pre_hbm.at[pl.ds(base + j * CB, CB)], SW[sel])
        return carry

    lax.fori_loop(0, NCH, chunk, 0)
    last = (NCH - 1) % 2
    pltpu.make_async_copy(BS[last], pre_hbm.at[pl.ds(0, CB)], SW[last]).wait()


def _sc_gather(hsa, hta, row, col):
    kern = functools.partial(
        pl.kernel,
        out_type=jax.ShapeDtypeStruct((E, DA), jnp.float32),
        mesh=plsc.VectorSubcoreMesh(core_axis_name="c", subcore_axis_name="s"),
        compiler_params=pltpu.CompilerParams(use_tc_tiling_on_sc=False),
        scratch_types=[
            pltpu.VMEM((EPW,), jnp.int32),
            pltpu.VMEM((EPW,), jnp.int32),
            pltpu.VMEM((CB, DA), jnp.float32),
            pltpu.VMEM((CB, DA), jnp.float32),
            pltpu.VMEM((CB, DA), jnp.float32),
            pltpu.VMEM((CB, DA), jnp.float32),
            pltpu.SemaphoreType.DMA,
            pltpu.SemaphoreType.DMA,
            pltpu.SemaphoreType.DMA,
            pltpu.SemaphoreType.DMA,
            pltpu.SemaphoreType.DMA,
            pltpu.SemaphoreType.DMA,
        ],
    )(_sc_gather_body)
    return kern(hsa, hta, row, col)


def _edge_body(pre_ref, ea_ref, w1e_ref, w1r_ref, b1_ref,
               w2_ref, b2_ref, out_ref):
    p = pre_ref[...]
    radial = jnp.sum(p[:, D:], axis=1, keepdims=True)
    x = (p[:, :D]
         + radial * w1r_ref[...]
         + jnp.dot(ea_ref[...], w1e_ref[...], preferred_element_type=jnp.float32)
         + b1_ref[...])
    m = x * jax.nn.sigmoid(x)
    y = jnp.dot(m, w2_ref[...], preferred_element_type=jnp.float32) + b2_ref[...]
    out_ref[...] = y * jax.nn.sigmoid(y)


def _edge_mlp(pre, ea, w1e, w1r, b1, w2, b2):
    be = 2000
    return pl.pallas_call(
        _edge_body,
        grid=(E // be,),
        in_specs=[
            pl.BlockSpec((be, DA), lambda i: (i, 0)),
            pl.BlockSpec((be, DE), lambda i: (i, 0)),
            pl.BlockSpec((DE, D), lambda i: (0, 0)),
            pl.BlockSpec((1, D), lambda i: (0, 0)),
            pl.BlockSpec((1, D), lambda i: (0, 0)),
            pl.BlockSpec((D, D), lambda i: (0, 0)),
            pl.BlockSpec((1, D), lambda i: (0, 0)),
        ],
        out_specs=pl.BlockSpec((be, D), lambda i: (i, 0)),
        out_shape=jax.ShapeDtypeStruct((E, D), jnp.float32),
    )(pre, ea, w1e, w1r, b1, w2, b2)


def _sc_scatter_body(mij_hbm, row_hbm, out_hbm,
                     idx0, idx1, mb0, mb1, zbuf, aggsp,
                     si0, si1, sm0, sm1):
    c = lax.axis_index("c")
    s = lax.axis_index("s")
    wid = s * NC + c
    base = wid * EPW

    IDX = (idx0, idx1)
    MB = (mb0, mb1)
    SI = (si0, si1)
    SM = (sm0, sm1)

    def issue_loads(j, sel):
        ebase = base + j * CB
        pltpu.async_copy(row_hbm.at[pl.ds(ebase, CB)], IDX[sel], SI[sel])
        pltpu.async_copy(mij_hbm.at[pl.ds(ebase, CB)], MB[sel], SM[sel])

    def zb(r, carry):
        for cc in range(D // 16):
            zbuf[r, pl.ds(cc * 16, 16)] = jnp.zeros((16,), jnp.float32)
        return carry

    lax.fori_loop(0, ZR, zb, 0)
    for k in range(RPT // ZR):
        pltpu.sync_copy(zbuf, aggsp.at[pl.ds(s * RPT + k * ZR, ZR)])
    plsc.subcore_barrier()

    issue_loads(0, 0)

    def chunk(j, carry):
        for sel in (0, 1):
            nsel = 1 - sel

            @pl.when(j % 2 == sel)
            def _():
                @pl.when(j + 1 < NCH)
                def _():
                    issue_loads(j + 1, nsel)

                pltpu.make_async_copy(
                    row_hbm.at[pl.ds(0, CB)], IDX[sel], SI[sel]).wait()
                pltpu.make_async_copy(
                    mij_hbm.at[pl.ds(0, CB)], MB[sel], SM[sel]).wait()
                pltpu.sync_copy(MB[sel], aggsp.at[IDX[sel]], add=True)
        return carry

    lax.fori_loop(0, NCH, chunk, 0)
    plsc.subcore_barrier()

    for k in range(RPT // ZR):
        r0 = s * RPT + k * ZR
        pltpu.sync_copy(aggsp.at[pl.ds(r0, ZR)], zbuf)
        pltpu.sync_copy(zbuf, out_hbm.at[c, pl.ds(r0, ZR)])


def _sc_scatter(mij, row):
    kern = functools.partial(
        pl.kernel,
        out_type=jax.ShapeDtypeStruct((NC, NP, D), jnp.float32),
        mesh=plsc.VectorSubcoreMesh(core_axis_name="c", subcore_axis_name="s"),
        scratch_types=[
            pltpu.VMEM((CB,), jnp.int32),
            pltpu.VMEM((CB,), jnp.int32),
            pltpu.VMEM((CB, D), jnp.float32),
            pltpu.VMEM((CB, D), jnp.float32),
            pltpu.VMEM((ZR, D), jnp.float32),
            pltpu.VMEM_SHARED((NP, D), jnp.float32),
            pltpu.SemaphoreType.DMA,
            pltpu.SemaphoreType.DMA,
            pltpu.SemaphoreType.DMA,
            pltpu.SemaphoreType.DMA,
        ],
    )(_sc_scatter_body)
    return kern(mij, row)


def _node_body(h_ref, agg_ref, w3h_ref, w3a_ref, b3_ref, w4_ref, b4_ref,
               out_ref):
    hb = h_ref[...]
    agg = agg_ref[0] + agg_ref[1]
    t = (jnp.dot(hb, w3h_ref[...], preferred_element_type=jnp.float32)
         + jnp.dot(agg, w3a_ref[...], preferred_element_type=jnp.float32)
         + b3_ref[...])
    t = t * jax.nn.sigmoid(t)
    out_ref[...] = (hb
                    + jnp.dot(t, w4_ref[...], preferred_element_type=jnp.float32)
                    + b4_ref[...])


def _node_mlp(h, agg2, w3h, w3a, b3, w4, b4):
    bn = 1000
    return pl.pallas_call(
        _node_body,
        grid=(N // bn,),
        in_specs=[
            pl.BlockSpec((bn, D), lambda i: (i, 0)),
            pl.BlockSpec((NC, bn, D), lambda i: (0, i, 0)),
            pl.BlockSpec((D, D), lambda i: (0, 0)),
            pl.BlockSpec((D, D), lambda i: (0, 0)),
            pl.BlockSpec((1, D), lambda i: (0, 0)),
            pl.BlockSpec((D, D), lambda i: (0, 0)),
            pl.BlockSpec((1, D), lambda i: (0, 0)),
        ],
        out_specs=pl.BlockSpec((bn, D), lambda i: (i, 0)),
        out_shape=jax.ShapeDtypeStruct((N, D), jnp.float32),
    )(h, agg2, w3h, w3a, b3, w4, b4)


def kernel(h, edge_index, coord, edge_attr, W1, b1, W2, b2, W3, b3, W4, b4):
    row = edge_index[0].astype(jnp.int32)
    col = edge_index[1].astype(jnp.int32)
    w1s = W1[:D]
    w1t = W1[D:2 * D]
    w1r = W1[2 * D:2 * D + 1]
    w1e = W1[2 * D + 1:]

    hs, ht = _project(h, w1s, w1t)
    n2 = jnp.sum(coord * coord, axis=1, keepdims=True)
    ones = jnp.ones((N, 1), jnp.float32)
    zpad = jnp.zeros((N, 11), jnp.float32)
    hsa = jnp.concatenate([hs, coord, ones, n2, zpad], axis=1)
    hta = jnp.concatenate([ht, -2.0 * coord, n2, ones, zpad], axis=1)
    pre = _sc_gather(hsa, hta, row, col)
    mij = _edge_mlp(pre, edge_attr, w1e, w1r,
                    b1.reshape(1, D), W2, b2.reshape(1, D))
    agg2 = _sc_scatter(mij, row)
    return _node_mlp(h, agg2, W3[:D], W3[D:], b3.reshape(1, D), W4,
                     b4.reshape(1, D))


# speedup vs baseline: 1.2044x; 1.0692x over previous
import functools

import jax
import jax.numpy as jnp
from jax import lax
from jax.experimental import pallas as pl
from jax.experimental.pallas import tpu as pltpu
from jax.experimental.pallas import tpu_sc as plsc

N = 10000
E = 320000
D = 128
DE = 16

NC = 2
NS = 16
NW = NC * NS
EPW = E // NW
CB = 80
NCH = EPW // CB
NP = 10240
RPT = NP // NS
ZR = 128


def _proj_body(h_ref, ws_ref, wt_ref, os_ref, ot_ref):
    x = h_ref[...]
    os_ref[...] = jnp.dot(x, ws_ref[...], preferred_element_type=jnp.float32)
    ot_ref[...] = jnp.dot(x, wt_ref[...], preferred_element_type=jnp.float32)


def _project(h, w1s, w1t):
    bn = 1000
    return pl.pallas_call(
        _proj_body,
        grid=(N // bn,),
        in_specs=[
            pl.BlockSpec((bn, D), lambda i: (i, 0)),
            pl.BlockSpec((D, D), lambda i: (0, 0)),
            pl.BlockSpec((D, D), lambda i: (0, 0)),
        ],
        out_specs=[
            pl.BlockSpec((bn, D), lambda i: (i, 0)),
            pl.BlockSpec((bn, D), lambda i: (i, 0)),
        ],
        out_shape=[
            jax.ShapeDtypeStruct((N, D), jnp.float32),
            jax.ShapeDtypeStruct((N, D), jnp.float32),
        ],
    )(h, w1s, w1t)


DA = D + 16


def _sc_gather_body(hs_hbm, ht_hbm, row_hbm, col_hbm,
                    pre_hbm, rad_hbm,
                    rowv, colv, bufs0, buft0, bufs1, buft1, ob0, ob1, rb0, rb1,
                    ss0, st0, ss1, st1, sw0, sw1, sr0, sr1):
    c = lax.axis_index("c")
    s = lax.axis_index("s")
    wid = s * NC + c
    base = wid * EPW

    pltpu.sync_copy(row_hbm.at[pl.ds(base, EPW)], rowv)
    pltpu.sync_copy(col_hbm.at[pl.ds(base, EPW)], colv)

    BS = (bufs0, bufs1)
    BT = (buft0, buft1)
    OB = (ob0, ob1)
    RB = (rb0, rb1)
    lanes16 = lax.iota(jnp.int32, 16)
    SS = (ss0, ss1)
    ST = (st0, st1)
    SW = (sw0, sw1)
    SR = (sr0, sr1)

    def issue_gathers(j, sel):
        off = j * CB
        pltpu.async_copy(hs_hbm.at[rowv.at[pl.ds(off, CB)]], BS[sel], SS[sel])
        pltpu.async_copy(ht_hbm.at[colv.at[pl.ds(off, CB)]], BT[sel], ST[sel])

    issue_gathers(0, 0)

    def chunk(j, carry):
        for sel in (0, 1):
            nsel = 1 - sel

            @pl.when(j % 2 == sel)
            def _():
                @pl.when(j >= 1)
                def _():
                    pltpu.make_async_copy(
                        OB[nsel], pre_hbm.at[pl.ds(0, CB)], SW[nsel]).wait()
                    pltpu.make_async_copy(
                        RB[nsel], rad_hbm.at[pl.ds(0, CB)], SR[nsel]).wait()

                @pl.when(j + 1 < NCH)
                def _():
                    issue_gathers(j + 1, nsel)

                pltpu.make_async_copy(
                    hs_hbm.at[rowv.at[pl.ds(0, CB)]], BS[sel], SS[sel]).wait()
                pltpu.make_async_copy(
                    ht_hbm.at[colv.at[pl.ds(0, CB)]], BT[sel], ST[sel]).wait()

                def add_body(g, carry2):
                    rv = jnp.zeros((16,), jnp.float32)
                    for k in range(16):
                        r = g * 16 + k
                        for cc in range(D // 16):
                            sl = pl.ds(cc * 16, 16)
                            OB[sel][r, sl] = BS[sel][r, sl] + BT[sel][r, sl]
                        v = (BS[sel][r, pl.ds(D, 16)]
                             + BT[sel][r, pl.ds(D, 16)])
                        rad = v[0] * v[0] + v[1] * v[1] + v[2] * v[2]
                        rv = jnp.where(lanes16 == k, rad, rv)
                    RB[sel][pl.ds(g * 16, 16)] = rv
                    return carry2

                lax.fori_loop(0, CB // 16, add_body, 0)
                pltpu.async_copy(
                    OB[sel], pre_hbm.at[pl.ds(base + j * CB, CB)], SW[sel])
                pltpu.async_copy(
                    RB[sel], rad_hbm.at[pl.ds(base + j * CB, CB)], SR[sel])
        return carry

    lax.fori_loop(0, NCH, chunk, 0)
    last = (NCH - 1) % 2
    pltpu.make_async_copy(OB[last], pre_hbm.at[pl.ds(0, CB)], SW[last]).wait()
    pltpu.make_async_copy(RB[last], rad_hbm.at[pl.ds(0, CB)], SR[last]).wait()


def _sc_gather(hsa, hta, row, col):
    kern = functools.partial(
        pl.kernel,
        out_type=[
            jax.ShapeDtypeStruct((E, D), jnp.float32),
            jax.ShapeDtypeStruct((E,), jnp.float32),
        ],
        mesh=plsc.VectorSubcoreMesh(core_axis_name="c", subcore_axis_name="s"),
        compiler_params=pltpu.CompilerParams(use_tc_tiling_on_sc=False),
        scratch_types=[
            pltpu.VMEM((EPW,), jnp.int32),
            pltpu.VMEM((EPW,), jnp.int32),
            pltpu.VMEM((CB, DA), jnp.float32),
            pltpu.VMEM((CB, DA), jnp.float32),
            pltpu.VMEM((CB, DA), jnp.float32),
            pltpu.VMEM((CB, DA), jnp.float32),
            pltpu.VMEM((CB, D), jnp.float32),
            pltpu.VMEM((CB, D), jnp.float32),
            pltpu.VMEM((CB,), jnp.float32),
            pltpu.VMEM((CB,), jnp.float32),
            pltpu.SemaphoreType.DMA,
            pltpu.SemaphoreType.DMA,
            pltpu.SemaphoreType.DMA,
            pltpu.SemaphoreType.DMA,
            pltpu.SemaphoreType.DMA,
            pltpu.SemaphoreType.DMA,
            pltpu.SemaphoreType.DMA,
            pltpu.SemaphoreType.DMA,
        ],
    )(_sc_gather_body)
    return kern(hsa, hta, row, col)


BE = 1280


def _edge_body(pre_ref, rad_ref, ea_ref, w1e_ref, w1r_ref, b1_ref,
               w2_ref, b2_ref, out_ref):
    rg = rad_ref[0]
    isub = lax.broadcasted_iota(jnp.int32, (BE, 1), 0)
    grp = lax.broadcasted_iota(jnp.int32, (BE, BE // 128), 1)
    onehot = jnp.where(isub // 128 == grp, 1.0, 0.0)
    y = jnp.dot(onehot, rg, preferred_element_type=jnp.float32)
    lane = lax.broadcasted_iota(jnp.int32, (BE, 128), 1)
    msk = jnp.where(lane == isub % 128, 1.0, 0.0)
    radial = jnp.sum(y * msk, axis=1, keepdims=True)
    x = (pre_ref[...]
         + radial * w1r_ref[...]
         + jnp.dot(ea_ref[...], w1e_ref[...], preferred_element_type=jnp.float32)
         + b1_ref[...])
    m = x * jax.nn.sigmoid(x)
    y2 = jnp.dot(m, w2_ref[...], preferred_element_type=jnp.float32) + b2_ref[...]
    out_ref[...] = y2 * jax.nn.sigmoid(y2)


def _edge_mlp(pre, rad2d, ea, w1e, w1r, b1, w2, b2):
    return pl.pallas_call(
        _edge_body,
        grid=(E // BE,),
        in_specs=[
            pl.BlockSpec((BE, D), lambda i: (i, 0)),
            pl.BlockSpec((1, BE // 128, 128), lambda i: (i, 0, 0)),
            pl.BlockSpec((BE, DE), lambda i: (i, 0)),
            pl.BlockSpec((DE, D), lambda i: (0, 0)),
            pl.BlockSpec((1, D), lambda i: (0, 0)),
            pl.BlockSpec((1, D), lambda i: (0, 0)),
            pl.BlockSpec((D, D), lambda i: (0, 0)),
            pl.BlockSpec((1, D), lambda i: (0, 0)),
        ],
        out_specs=pl.BlockSpec((BE, D), lambda i: (i, 0)),
        out_shape=jax.ShapeDtypeStruct((E, D), jnp.float32),
    )(pre, rad2d, ea, w1e, w1r, b1, w2, b2)


def _sc_scatter_body(mij_hbm, row_hbm, out_hbm,
                     idx0, idx1, mb0, mb1, zbuf, aggsp,
                     si0, si1, sm0, sm1):
    c = lax.axis_index("c")
    s = lax.axis_index("s")
    wid = s * NC + c
    base = wid * EPW

    IDX = (idx0, idx1)
    MB = (mb0, mb1)
    SI = (si0, si1)
    SM = (sm0, sm1)

    def issue_loads(j, sel):
        ebase = base + j * CB
        pltpu.async_copy(row_hbm.at[pl.ds(ebase, CB)], IDX[sel], SI[sel])
        pltpu.async_copy(mij_hbm.at[pl.ds(ebase, CB)], MB[sel], SM[sel])

    def zb(r, carry):
        for cc in range(D // 16):
            zbuf[r, pl.ds(cc * 16, 16)] = jnp.zeros((16,), jnp.float32)
        return carry

    lax.fori_loop(0, ZR, zb, 0)
    for k in range(RPT // ZR):
        pltpu.sync_copy(zbuf, aggsp.at[pl.ds(s * RPT + k * ZR, ZR)])
    plsc.subcore_barrier()

    issue_loads(0, 0)

    def chunk(j, carry):
        for sel in (0, 1):
            nsel = 1 - sel

            @pl.when(j % 2 == sel)
            def _():
                @pl.when(j + 1 < NCH)
                def _():
                    issue_loads(j + 1, nsel)

                pltpu.make_async_copy(
                    row_hbm.at[pl.ds(0, CB)], IDX[sel], SI[sel]).wait()
                pltpu.make_async_copy(
                    mij_hbm.at[pl.ds(0, CB)], MB[sel], SM[sel]).wait()
                pltpu.sync_copy(MB[sel], aggsp.at[IDX[sel]], add=True)
        return carry

    lax.fori_loop(0, NCH, chunk, 0)
    plsc.subcore_barrier()

    for k in range(RPT // ZR):
        r0 = s * RPT + k * ZR
        pltpu.sync_copy(aggsp.at[pl.ds(r0, ZR)], zbuf)
        pltpu.sync_copy(zbuf, out_hbm.at[c, pl.ds(r0, ZR)])


def _sc_scatter(mij, row):
    kern = functools.partial(
        pl.kernel,
        out_type=jax.ShapeDtypeStruct((NC, NP, D), jnp.float32),
        mesh=plsc.VectorSubcoreMesh(core_axis_name="c", subcore_axis_name="s"),
        scratch_types=[
            pltpu.VMEM((CB,), jnp.int32),
            pltpu.VMEM((CB,), jnp.int32),
            pltpu.VMEM((CB, D), jnp.float32),
            pltpu.VMEM((CB, D), jnp.float32),
            pltpu.VMEM((ZR, D), jnp.float32),
            pltpu.VMEM_SHARED((NP, D), jnp.float32),
            pltpu.SemaphoreType.DMA,
            pltpu.SemaphoreType.DMA,
            pltpu.SemaphoreType.DMA,
            pltpu.SemaphoreType.DMA,
        ],
    )(_sc_scatter_body)
    return kern(mij, row)


def _node_body(h_ref, agg_ref, w3h_ref, w3a_ref, b3_ref, w4_ref, b4_ref,
               out_ref):
    hb = h_ref[...]
    agg = agg_ref[0] + agg_ref[1]
    t = (jnp.dot(hb, w3h_ref[...], preferred_element_type=jnp.float32)
         + jnp.dot(agg, w3a_ref[...], preferred_element_type=jnp.float32)
         + b3_ref[...])
    t = t * jax.nn.sigmoid(t)
    out_ref[...] = (hb
                    + jnp.dot(t, w4_ref[...], preferred_element_type=jnp.float32)
                    + b4_ref[...])


def _node_mlp(h, agg2, w3h, w3a, b3, w4, b4):
    bn = 1000
    return pl.pallas_call(
        _node_body,
        grid=(N // bn,),
        in_specs=[
            pl.BlockSpec((bn, D), lambda i: (i, 0)),
            pl.BlockSpec((NC, bn, D), lambda i: (0, i, 0)),
            pl.BlockSpec((D, D), lambda i: (0, 0)),
            pl.BlockSpec((D, D), lambda i: (0, 0)),
            pl.BlockSpec((1, D), lambda i: (0, 0)),
            pl.BlockSpec((D, D), lambda i: (0, 0)),
            pl.BlockSpec((1, D), lambda i: (0, 0)),
        ],
        out_specs=pl.BlockSpec((bn, D), lambda i: (i, 0)),
        out_shape=jax.ShapeDtypeStruct((N, D), jnp.float32),
    )(h, agg2, w3h, w3a, b3, w4, b4)


def kernel(h, edge_index, coord, edge_attr, W1, b1, W2, b2, W3, b3, W4, b4):
    row = edge_index[0].astype(jnp.int32)
    col = edge_index[1].astype(jnp.int32)
    w1s = W1[:D]
    w1t = W1[D:2 * D]
    w1r = W1[2 * D:2 * D + 1]
    w1e = W1[2 * D + 1:]

    hs, ht = _project(h, w1s, w1t)
    zpad = jnp.zeros((N, 13), jnp.float32)
    hsa = jnp.concatenate([hs, coord, zpad], axis=1)
    hta = jnp.concatenate([ht, -coord, zpad], axis=1)
    pre, rad = _sc_gather(hsa, hta, row, col)
    mij = _edge_mlp(pre, rad.reshape(E // BE, BE // 128, 128), edge_attr, w1e, w1r,
                    b1.reshape(1, D), W2, b2.reshape(1, D))
    agg2 = _sc_scatter(mij, row)
    return _node_mlp(h, agg2, W3[:D], W3[D:], b3.reshape(1, D), W4,
                     b4.reshape(1, D))


# speedup vs baseline: 1.3005x; 1.0798x over previous
import functools

import jax
import jax.numpy as jnp
from jax import lax
from jax.experimental import pallas as pl
from jax.experimental.pallas import tpu as pltpu
from jax.experimental.pallas import tpu_sc as plsc

N = 10000
E = 320000
D = 128
DE = 16

NC = 2
NS = 16
NW = NC * NS
EPW = E // NW
CB = 80
NCH = EPW // CB
NP = 10240
RPT = NP // NS
ZR = 128


def _proj_body(h_ref, ws_ref, wt_ref, os_ref, ot_ref):
    x = h_ref[...]
    os_ref[...] = jnp.dot(x, ws_ref[...], preferred_element_type=jnp.float32)
    ot_ref[...] = jnp.dot(x, wt_ref[...], preferred_element_type=jnp.float32)


def _project(h, w1s, w1t):
    bn = 1000
    return pl.pallas_call(
        _proj_body,
        grid=(N // bn,),
        in_specs=[
            pl.BlockSpec((bn, D), lambda i: (i, 0)),
            pl.BlockSpec((D, D), lambda i: (0, 0)),
            pl.BlockSpec((D, D), lambda i: (0, 0)),
        ],
        out_specs=[
            pl.BlockSpec((bn, D), lambda i: (i, 0)),
            pl.BlockSpec((bn, D), lambda i: (i, 0)),
        ],
        out_shape=[
            jax.ShapeDtypeStruct((N, D), jnp.float32),
            jax.ShapeDtypeStruct((N, D), jnp.float32),
        ],
    )(h, w1s, w1t)


DA = D + 16


def _sc_gather_body(hs_hbm, ht_hbm, row_hbm, col_hbm,
                    pre_hbm, rad_hbm,
                    rowv, colv, bufs0, buft0, bufs1, buft1, ob0, ob1, radv,
                    ss0, st0, ss1, st1, sw0, sw1):
    c = lax.axis_index("c")
    s = lax.axis_index("s")
    wid = s * NC + c
    base = wid * EPW

    pltpu.sync_copy(row_hbm.at[pl.ds(base, EPW)], rowv)
    pltpu.sync_copy(col_hbm.at[pl.ds(base, EPW)], colv)

    BS = (bufs0, bufs1)
    BT = (buft0, buft1)
    OB = (ob0, ob1)
    lanes16 = lax.iota(jnp.int32, 16)
    SS = (ss0, ss1)
    ST = (st0, st1)
    SW = (sw0, sw1)

    def issue_gathers(j, sel):
        off = j * CB
        pltpu.async_copy(hs_hbm.at[rowv.at[pl.ds(off, CB)]], BS[sel], SS[sel])
        pltpu.async_copy(ht_hbm.at[colv.at[pl.ds(off, CB)]], BT[sel], ST[sel])

    issue_gathers(0, 0)

    def chunk(j, carry):
        for sel in (0, 1):
            nsel = 1 - sel

            @pl.when(j % 2 == sel)
            def _():
                @pl.when(j >= 1)
                def _():
                    pltpu.make_async_copy(
                        OB[nsel], pre_hbm.at[pl.ds(0, CB)], SW[nsel]).wait()

                @pl.when(j + 1 < NCH)
                def _():
                    issue_gathers(j + 1, nsel)

                pltpu.make_async_copy(
                    hs_hbm.at[rowv.at[pl.ds(0, CB)]], BS[sel], SS[sel]).wait()
                pltpu.make_async_copy(
                    ht_hbm.at[colv.at[pl.ds(0, CB)]], BT[sel], ST[sel]).wait()

                def add_body(g, carry2):
                    rv = jnp.zeros((16,), jnp.float32)
                    for k in range(16):
                        r = g * 16 + k
                        for cc in range(D // 16):
                            sl = pl.ds(cc * 16, 16)
                            OB[sel][r, sl] = BS[sel][r, sl] + BT[sel][r, sl]
                        v = (BS[sel][r, pl.ds(D, 16)]
                             + BT[sel][r, pl.ds(D, 16)])
                        rad = v[0] * v[0] + v[1] * v[1] + v[2] * v[2]
                        rv = jnp.where(lanes16 == k, rad, rv)
                    radv[pl.ds(j * CB + g * 16, 16)] = rv
                    return carry2

                lax.fori_loop(0, CB // 16, add_body, 0)
                pltpu.async_copy(
                    OB[sel], pre_hbm.at[pl.ds(base + j * CB, CB)], SW[sel])
        return carry

    lax.fori_loop(0, NCH, chunk, 0)
    pltpu.sync_copy(radv, rad_hbm.at[pl.ds(base, EPW)])
    last = (NCH - 1) % 2
    pltpu.make_async_copy(OB[last], pre_hbm.at[pl.ds(0, CB)], SW[last]).wait()


def _sc_gather(hsa, hta, row, col):
    kern = functools.partial(
        pl.kernel,
        out_type=[
            jax.ShapeDtypeStruct((E, D), jnp.float32),
            jax.ShapeDtypeStruct((E,), jnp.float32),
        ],
        mesh=plsc.VectorSubcoreMesh(core_axis_name="c", subcore_axis_name="s"),
        compiler_params=pltpu.CompilerParams(use_tc_tiling_on_sc=False),
        scratch_types=[
            pltpu.VMEM((EPW,), jnp.int32),
            pltpu.VMEM((EPW,), jnp.int32),
            pltpu.VMEM((CB, DA), jnp.float32),
            pltpu.VMEM((CB, DA), jnp.float32),
            pltpu.VMEM((CB, DA), jnp.float32),
            pltpu.VMEM((CB, DA), jnp.float32),
            pltpu.VMEM((CB, D), jnp.float32),
            pltpu.VMEM((CB, D), jnp.float32),
            pltpu.VMEM((EPW,), jnp.float32),
            pltpu.SemaphoreType.DMA,
            pltpu.SemaphoreType.DMA,
            pltpu.SemaphoreType.DMA,
            pltpu.SemaphoreType.DMA,
            pltpu.SemaphoreType.DMA,
            pltpu.SemaphoreType.DMA,
        ],
    )(_sc_gather_body)
    return kern(hsa, hta, row, col)


BE = 2560


def _edge_body(pre_ref, rad_ref, ea_ref, w1e_ref, w1r_ref, b1_ref,
               w2_ref, b2_ref, out_ref):
    rg = rad_ref[0]
    isub = lax.broadcasted_iota(jnp.int32, (BE, 1), 0)
    grp = lax.broadcasted_iota(jnp.int32, (BE, BE // 128), 1)
    onehot = jnp.where(isub // 128 == grp, 1.0, 0.0)
    y = jnp.dot(onehot, rg, preferred_element_type=jnp.float32)
    lane = lax.broadcasted_iota(jnp.int32, (BE, 128), 1)
    msk = jnp.where(lane == isub % 128, 1.0, 0.0)
    radial = jnp.sum(y * msk, axis=1, keepdims=True)
    x = (pre_ref[...]
         + radial * w1r_ref[...]
         + jnp.dot(ea_ref[...], w1e_ref[...], preferred_element_type=jnp.float32)
         + b1_ref[...])
    m = x * jax.nn.sigmoid(x)
    y2 = jnp.dot(m, w2_ref[...], preferred_element_type=jnp.float32) + b2_ref[...]
    out_ref[...] = y2 * jax.nn.sigmoid(y2)


def _edge_mlp(pre, rad2d, ea, w1e, w1r, b1, w2, b2):
    return pl.pallas_call(
        _edge_body,
        grid=(E // BE,),
        in_specs=[
            pl.BlockSpec((BE, D), lambda i: (i, 0)),
            pl.BlockSpec((1, BE // 128, 128), lambda i: (i, 0, 0)),
            pl.BlockSpec((BE, DE), lambda i: (i, 0)),
            pl.BlockSpec((DE, D), lambda i: (0, 0)),
            pl.BlockSpec((1, D), lambda i: (0, 0)),
            pl.BlockSpec((1, D), lambda i: (0, 0)),
            pl.BlockSpec((D, D), lambda i: (0, 0)),
            pl.BlockSpec((1, D), lambda i: (0, 0)),
        ],
        out_specs=pl.BlockSpec((BE, D), lambda i: (i, 0)),
        out_shape=jax.ShapeDtypeStruct((E, D), jnp.float32),
    )(pre, rad2d, ea, w1e, w1r, b1, w2, b2)


def _sc_scatter_body(mij_hbm, row_hbm, out_hbm,
                     idx0, idx1, mb0, mb1, zbuf, aggsp,
                     si0, si1, sm0, sm1):
    c = lax.axis_index("c")
    s = lax.axis_index("s")
    wid = s * NC + c
    base = wid * EPW

    IDX = (idx0, idx1)
    MB = (mb0, mb1)
    SI = (si0, si1)
    SM = (sm0, sm1)

    def issue_loads(j, sel):
        ebase = base + j * CB
        pltpu.async_copy(row_hbm.at[pl.ds(ebase, CB)], IDX[sel], SI[sel])
        pltpu.async_copy(mij_hbm.at[pl.ds(ebase, CB)], MB[sel], SM[sel])

    def zb(r, carry):
        for cc in range(D // 16):
            zbuf[r, pl.ds(cc * 16, 16)] = jnp.zeros((16,), jnp.float32)
        return carry

    lax.fori_loop(0, ZR, zb, 0)
    for k in range(RPT // ZR):
        pltpu.sync_copy(zbuf, aggsp.at[pl.ds(s * RPT + k * ZR, ZR)])
    plsc.subcore_barrier()

    issue_loads(0, 0)

    def chunk(j, carry):
        for sel in (0, 1):
            nsel = 1 - sel

            @pl.when(j % 2 == sel)
            def _():
                @pl.when(j + 1 < NCH)
                def _():
                    issue_loads(j + 1, nsel)

                pltpu.make_async_copy(
                    row_hbm.at[pl.ds(0, CB)], IDX[sel], SI[sel]).wait()
                pltpu.make_async_copy(
                    mij_hbm.at[pl.ds(0, CB)], MB[sel], SM[sel]).wait()
                pltpu.sync_copy(MB[sel], aggsp.at[IDX[sel]], add=True)
        return carry

    lax.fori_loop(0, NCH, chunk, 0)
    plsc.subcore_barrier()

    for k in range(RPT // ZR):
        r0 = s * RPT + k * ZR
        pltpu.sync_copy(aggsp.at[pl.ds(r0, ZR)], zbuf)
        pltpu.sync_copy(zbuf, out_hbm.at[c, pl.ds(r0, ZR)])


def _sc_scatter(mij, row):
    kern = functools.partial(
        pl.kernel,
        out_type=jax.ShapeDtypeStruct((NC, NP, D), jnp.float32),
        mesh=plsc.VectorSubcoreMesh(core_axis_name="c", subcore_axis_name="s"),
        scratch_types=[
            pltpu.VMEM((CB,), jnp.int32),
            pltpu.VMEM((CB,), jnp.int32),
            pltpu.VMEM((CB, D), jnp.float32),
            pltpu.VMEM((CB, D), jnp.float32),
            pltpu.VMEM((ZR, D), jnp.float32),
            pltpu.VMEM_SHARED((NP, D), jnp.float32),
            pltpu.SemaphoreType.DMA,
            pltpu.SemaphoreType.DMA,
            pltpu.SemaphoreType.DMA,
            pltpu.SemaphoreType.DMA,
        ],
    )(_sc_scatter_body)
    return kern(mij, row)


def _node_body(h_ref, agg_ref, w3h_ref, w3a_ref, b3_ref, w4_ref, b4_ref,
               out_ref):
    hb = h_ref[...]
    agg = agg_ref[0] + agg_ref[1]
    t = (jnp.dot(hb, w3h_ref[...], preferred_element_type=jnp.float32)
         + jnp.dot(agg, w3a_ref[...], preferred_element_type=jnp.float32)
         + b3_ref[...])
    t = t * jax.nn.sigmoid(t)
    out_ref[...] = (hb
                    + jnp.dot(t, w4_ref[...], preferred_element_type=jnp.float32)
                    + b4_ref[...])


def _node_mlp(h, agg2, w3h, w3a, b3, w4, b4):
    bn = 1000
    return pl.pallas_call(
        _node_body,
        grid=(N // bn,),
        in_specs=[
            pl.BlockSpec((bn, D), lambda i: (i, 0)),
            pl.BlockSpec((NC, bn, D), lambda i: (0, i, 0)),
            pl.BlockSpec((D, D), lambda i: (0, 0)),
            pl.BlockSpec((D, D), lambda i: (0, 0)),
            pl.BlockSpec((1, D), lambda i: (0, 0)),
            pl.BlockSpec((D, D), lambda i: (0, 0)),
            pl.BlockSpec((1, D), lambda i: (0, 0)),
        ],
        out_specs=pl.BlockSpec((bn, D), lambda i: (i, 0)),
        out_shape=jax.ShapeDtypeStruct((N, D), jnp.float32),
    )(h, agg2, w3h, w3a, b3, w4, b4)


def kernel(h, edge_index, coord, edge_attr, W1, b1, W2, b2, W3, b3, W4, b4):
    row = edge_index[0].astype(jnp.int32)
    col = edge_index[1].astype(jnp.int32)
    w1s = W1[:D]
    w1t = W1[D:2 * D]
    w1r = W1[2 * D:2 * D + 1]
    w1e = W1[2 * D + 1:]

    hs, ht = _project(h, w1s, w1t)
    zpad = jnp.zeros((N, 13), jnp.float32)
    hsa = jnp.concatenate([hs, coord, zpad], axis=1)
    hta = jnp.concatenate([ht, -coord, zpad], axis=1)
    pre, rad = _sc_gather(hsa, hta, row, col)
    mij = _edge_mlp(pre, rad.reshape(E // BE, BE // 128, 128), edge_attr, w1e, w1r,
                    b1.reshape(1, D), W2, b2.reshape(1, D))
    agg2 = _sc_scatter(mij, row)
    return _node_mlp(h, agg2, W3[:D], W3[D:], b3.reshape(1, D), W4,
                     b4.reshape(1, D))
